# baseline (speedup 1.0000x reference)
import jax
import jax.numpy as jnp
from jax import lax
from jax.experimental import pallas as pl
from jax.experimental.pallas import tpu as pltpu
from jax.experimental.pallas import tpu_sc as plsc

N = 10000
D = 128
H = 64
E = 320000
NC = 2
NS = 16
NW = NC * NS
EPW = 10240
EPAD = EPW * NW
B = 128
NB = EPW // B
NIB = NB
EROWS = NW * NB
NPAD = 10112
RPS = NPAD // NS
ZR = RPS // 2
U = 4

_f32 = jnp.float32
_mesh = plsc.VectorSubcoreMesh(core_axis_name="c", subcore_axis_name="s")
_sc_params = pltpu.CompilerParams(use_tc_tiling_on_sc=False)


def _worker_ids(idxb):
    c = lax.axis_index("c")
    s = lax.axis_index("s")
    w = c * NS + s
    lane = lax.iota(jnp.int32, 16)
    for k in range(NIB // 16):
        idxb[pl.ds(k * 16, 16)] = w * NB + k * 16 + lane
    return c, s


def _fetch_edges(edg, idxb, pckv, srcv, dstv, sem):
    pltpu.async_copy(edg.at[idxb], pckv, sem)
    pltpu.make_async_copy(edg.at[idxb], pckv, sem).wait()

    def unpack(j, carry):
        for k in range(B // 16):
            v = pckv[j, pl.ds(k * 16, 16)]
            if srcv is not None:
                srcv[j, pl.ds(k * 16, 16)] = lax.shift_right_logical(v, 16)
            dstv[j, pl.ds(k * 16, 16)] = lax.bitwise_and(v, 0xFFFF)
        return carry

    lax.fori_loop(0, NB, unpack, 0)


def _make_counts():
    out_type = jax.ShapeDtypeStruct((3, NC, NPAD), _f32)
    scratch = [
        pltpu.VMEM_SHARED((NPAD,), _f32),
        pltpu.VMEM((NIB,), jnp.int32),
        pltpu.VMEM((NIB, B), jnp.int32),
        pltpu.VMEM((NB, B), jnp.int32),
        pltpu.VMEM((640,), _f32),
        pltpu.VMEM((B,), _f32),
        pltpu.SemaphoreType.DMA,
    ]

    def body(er, ea, ev, cnts, cacc, idxb, pckv, dstv, zct, ones, sem):
        c, s = _worker_ids(idxb)
        z16 = jnp.zeros((16,), _f32)
        one16 = jnp.ones((16,), _f32)

        def fillz(i, carry):
            zct[pl.ds(i * 16, 16)] = z16
            return carry

        lax.fori_loop(0, 640 // 16, fillz, 0)
        for i in range(B // 16):
            ones[pl.ds(i * 16, 16)] = one16

        for t, edg in enumerate((er, ea, ev)):
            pltpu.sync_copy(zct.at[pl.ds(0, RPS)], cacc.at[pl.ds(s * RPS, RPS)])
            plsc.subcore_barrier()
            _fetch_edges(edg, idxb, pckv, None, dstv, sem)

            def step(j, carry):
                pltpu.sync_copy(ones, cacc.at[dstv.at[j]], add=True)
                return carry

            lax.fori_loop(0, NB, step, 0)
            plsc.subcore_barrier()
            pltpu.sync_copy(cacc.at[pl.ds(s * RPS, RPS)],
                            cnts.at[t, c, pl.ds(s * RPS, RPS)])

    return pl.kernel(body, out_type=(out_type,), mesh=_mesh,
                     scratch_types=scratch, compiler_params=_sc_params)


def _make_segsum():
    out_type = jax.ShapeDtypeStruct((3, NC, NPAD, H), _f32)
    scratch = [
        pltpu.VMEM_SHARED((NPAD, H), _f32),
        pltpu.VMEM((NIB,), jnp.int32),
        pltpu.VMEM((NIB, B), jnp.int32),
        pltpu.VMEM((NB, B), jnp.int32),
        pltpu.VMEM((NB, B), jnp.int32),
        [pltpu.VMEM((B, H), _f32)] * U,
        pltpu.VMEM((ZR, H), _f32),
        [pltpu.SemaphoreType.DMA] * U,
        [pltpu.SemaphoreType.DMA] * U,
    ]

    def body(tr, ta, tv, er, ea, ev, sums,
             acc, idxb, pckv, srcv, dstv, rows, zac, gsems, ssems):
        c, s = _worker_ids(idxb)
        z16 = jnp.zeros((16,), _f32)

        def fill(i, carry):
            for q in range(H // 16):
                zac[i, pl.ds(q * 16, 16)] = z16
            return carry

        lax.fori_loop(0, ZR, fill, 0)

        for t, (tbl, edg) in enumerate(((tr, er), (ta, ea), (tv, ev))):
            pltpu.sync_copy(zac, acc.at[pl.ds(s * RPS, ZR)])
            pltpu.sync_copy(zac, acc.at[pl.ds(s * RPS + ZR, ZR)])
            plsc.subcore_barrier()
            _fetch_edges(edg, idxb, pckv, srcv, dstv, gsems[0])

            def step(g, carry):
                j = g * U
                for u in range(U):
                    pltpu.async_copy(tbl.at[srcv.at[j + u]], rows[u], gsems[u])
                for u in range(U):
                    pltpu.make_async_copy(tbl.at[srcv.at[j + u]],
                                          rows[u], gsems[u]).wait()
                    pltpu.async_copy(rows[u], acc.at[dstv.at[j + u]],
                                     ssems[u], add=True)
                for u in range(U):
                    pltpu.make_async_copy(rows[u], acc.at[dstv.at[j + u]],
                                          ssems[u]).wait()
                return carry

            lax.fori_loop(0, NB // U, step, 0)
            plsc.subcore_barrier()
            pltpu.sync_copy(acc.at[pl.ds(s * RPS, RPS)],
                            sums.at[t, c, pl.ds(s * RPS, RPS)])

    return pl.kernel(body, out_type=(out_type,), mesh=_mesh,
                     scratch_types=scratch, compiler_params=_sc_params)


_counts_sc = _make_counts()
_segsum_sc = _make_segsum()


R = 1000
GRID = N // R


def _dot(a, b):
    return jnp.dot(a, b, preferred_element_type=_f32)


def _project3(xu, xi, w_r, w_a, w_v):
    def body(xu_ref, xi_ref, wr_ref, wa_ref, wv_ref, tr_ref, ta_ref, tv_ref):
        tr_ref[...] = _dot(xu_ref[...], wr_ref[...])
        xir = xi_ref[...]
        ta_ref[...] = _dot(xir, wa_ref[...])
        tv_ref[...] = _dot(xir, wv_ref[...])

    return pl.pallas_call(
        body,
        grid=(GRID,),
        in_specs=[
            pl.BlockSpec((R, D), lambda i: (i, 0)),
            pl.BlockSpec((R, D), lambda i: (i, 0)),
            pl.BlockSpec((D, H), lambda i: (0, 0)),
            pl.BlockSpec((D, H), lambda i: (0, 0)),
            pl.BlockSpec((D, H), lambda i: (0, 0)),
        ],
        out_specs=[pl.BlockSpec((R, H), lambda i: (i, 0))] * 3,
        out_shape=[jax.ShapeDtypeStruct((N, H), _f32)] * 3,
    )(xu, xi, w_r, w_a, w_v)


def _combine1(sums, cnts, xu, xi, w1rr, w1ar, w1vr, b1r, b1a, b1v,
              w2rl, w2al, w2vl):
    def body(s_ref, c_ref, xu_ref, xi_ref, w1rr_ref, w1ar_ref, w1vr_ref,
             b1r_ref, b1a_ref, b1v_ref, w2rl_ref, w2al_ref, w2vl_ref,
             u1_ref, i1_ref, t2r_ref, t2a_ref, t2v_ref):
        sr = s_ref[0, 0] + s_ref[0, 1]
        sa = s_ref[1, 0] + s_ref[1, 1]
        sv = s_ref[2, 0] + s_ref[2, 1]
        cr = c_ref[0, 0] + c_ref[0, 1]
        ca = c_ref[1, 0] + c_ref[1, 1]
        cv = c_ref[2, 0] + c_ref[2, 1]
        item = (sr / jnp.maximum(cr, 1.0) + b1r_ref[...]
                + sa / jnp.maximum(ca, 1.0) + b1a_ref[...]
                + _dot(xi_ref[...], w1rr_ref[...] + w1ar_ref[...]))
        user = (sv / jnp.maximum(cv, 1.0) + b1v_ref[...]
                + _dot(xu_ref[...], w1vr_ref[...]))
        item = jnp.maximum(item, 0.0)
        user = jnp.maximum(user, 0.0)
        u1_ref[...] = user
        i1_ref[...] = item
        t2r_ref[...] = _dot(user, w2rl_ref[...])
        t2a_ref[...] = _dot(item, w2al_ref[...])
        t2v_ref[...] = _dot(item, w2vl_ref[...])

    return pl.pallas_call(
        body,
        grid=(GRID,),
        in_specs=[
            pl.BlockSpec((3, NC, R, H), lambda i: (0, 0, i, 0)),
            pl.BlockSpec((3, NC, R, 1), lambda i: (0, 0, i, 0)),
            pl.BlockSpec((R, D), lambda i: (i, 0)),
            pl.BlockSpec((R, D), lambda i: (i, 0)),
            pl.BlockSpec((D, H), lambda i: (0, 0)),
            pl.BlockSpec((D, H), lambda i: (0, 0)),
            pl.BlockSpec((D, H), lambda i: (0, 0)),
            pl.BlockSpec((1, H), lambda i: (0, 0)),
            pl.BlockSpec((1, H), lambda i: (0, 0)),
            pl.BlockSpec((1, H), lambda i: (0, 0)),
            pl.BlockSpec((H, H), lambda i: (0, 0)),
            pl.BlockSpec((H, H), lambda i: (0, 0)),
            pl.BlockSpec((H, H), lambda i: (0, 0)),
        ],
        out_specs=[pl.BlockSpec((R, H), lambda i: (i, 0))] * 5,
        out_shape=[jax.ShapeDtypeStruct((N, H), _f32)] * 5,
    )(sums, cnts, xu, xi, w1rr, w1ar, w1vr, b1r, b1a, b1v, w2rl, w2al, w2vl)


def _combine2(sums, cnts, u1, i1, w2rr, w2ar, w2vr, b2r, b2a, b2v):
    def body(s_ref, c_ref, u1_ref, i1_ref, w2rr_ref, w2ar_ref, w2vr_ref,
             b2r_ref, b2a_ref, b2v_ref, u2_ref, i2_ref):
        sr = s_ref[0, 0] + s_ref[0, 1]
        sa = s_ref[1, 0] + s_ref[1, 1]
        sv = s_ref[2, 0] + s_ref[2, 1]
        cr = c_ref[0, 0] + c_ref[0, 1]
        ca = c_ref[1, 0] + c_ref[1, 1]
        cv = c_ref[2, 0] + c_ref[2, 1]
        i2_ref[...] = (sr / jnp.maximum(cr, 1.0) + b2r_ref[...]
                       + sa / jnp.maximum(ca, 1.0) + b2a_ref[...]
                       + _dot(i1_ref[...], w2rr_ref[...] + w2ar_ref[...]))
        u2_ref[...] = (sv / jnp.maximum(cv, 1.0) + b2v_ref[...]
                       + _dot(u1_ref[...], w2vr_ref[...]))

    return pl.pallas_call(
        body,
        grid=(GRID,),
        in_specs=[
            pl.BlockSpec((3, NC, R, H), lambda i: (0, 0, i, 0)),
            pl.BlockSpec((3, NC, R, 1), lambda i: (0, 0, i, 0)),
            pl.BlockSpec((R, H), lambda i: (i, 0)),
            pl.BlockSpec((R, H), lambda i: (i, 0)),
            pl.BlockSpec((H, H), lambda i: (0, 0)),
            pl.BlockSpec((H, H), lambda i: (0, 0)),
            pl.BlockSpec((H, H), lambda i: (0, 0)),
            pl.BlockSpec((1, H), lambda i: (0, 0)),
            pl.BlockSpec((1, H), lambda i: (0, 0)),
            pl.BlockSpec((1, H), lambda i: (0, 0)),
        ],
        out_specs=[pl.BlockSpec((R, H), lambda i: (i, 0))] * 2,
        out_shape=[jax.ShapeDtypeStruct((N, H), _f32)] * 2,
    )(sums, cnts, u1, i1, w2rr, w2ar, w2vr, b2r, b2a, b2v)


def kernel(x_user, x_item, edge_reviews, edge_rev_reviews, edge_also_bought,
           W1r_l, b1r, W1r_r, W1v_l, b1v, W1v_r, W1a_l, b1a, W1a_r,
           W2r_l, b2r, W2r_r, W2v_l, b2v, W2v_r, W2a_l, b2a, W2a_r):
    def _prep(e):
        e = e.astype(jnp.int32)
        src = jnp.concatenate([e[0], jnp.zeros((EPAD - E,), jnp.int32)])
        dst = jnp.concatenate([e[1], jnp.full((EPAD - E,), NPAD - 1,
                                              jnp.int32)])
        return ((src << 16) | dst).reshape(EROWS, B)

    er = _prep(edge_reviews)
    ev = _prep(edge_rev_reviews)
    ea = _prep(edge_also_bought)

    (cnts,) = _counts_sc(er, ea, ev)
    cnts = cnts.reshape(3, NC, NPAD, 1)
    t1r, t1a, t1v = _project3(x_user, x_item, W1r_l, W1a_l, W1v_l)
    (sums1,) = _segsum_sc(t1r, t1a, t1v, er, ea, ev)
    u1, i1, t2r, t2a, t2v = _combine1(
        sums1, cnts, x_user, x_item, W1r_r, W1a_r, W1v_r,
        b1r.reshape(1, H), b1a.reshape(1, H), b1v.reshape(1, H),
        W2r_l, W2a_l, W2v_l)
    (sums2,) = _segsum_sc(t2r, t2a, t2v, er, ea, ev)
    u2, i2 = _combine2(
        sums2, cnts, u1, i1, W2r_r, W2a_r, W2v_r,
        b2r.reshape(1, H), b2a.reshape(1, H), b2v.reshape(1, H))
    return (u2, i2)


# speedup vs baseline: 2.5532x; 2.5532x over previous
import jax
import jax.numpy as jnp
from jax import lax
from jax.experimental import pallas as pl
from jax.experimental.pallas import tpu as pltpu
from jax.experimental.pallas import tpu_sc as plsc

N = 10000
D = 128
H = 64
E = 320000
NC = 2
NS = 16
NW = NC * NS
EPW = E // NW
B = 80
NB = EPW // B
NIB = 128
EROWS = 4008
NPAD = 10112
RPS = NPAD // NS
ZR = RPS // 2
U = 5

_f32 = jnp.float32
_mesh = plsc.VectorSubcoreMesh(core_axis_name="c", subcore_axis_name="s")
_sc_params = pltpu.CompilerParams(use_tc_tiling_on_sc=False)


def _worker_ids(idxb):
    c = lax.axis_index("c")
    s = lax.axis_index("s")
    w = c * NS + s
    lane = lax.iota(jnp.int32, 16)
    for k in range(NIB // 16):
        idxb[pl.ds(k * 16, 16)] = w * NB + k * 16 + lane
    return c, s


def _fetch_edges(edg, idxb, pckv, srcv, dstv, sem):
    pltpu.async_copy(edg.at[idxb], pckv, sem)
    pltpu.make_async_copy(edg.at[idxb], pckv, sem).wait()

    def unpack(j, carry):
        for k in range(B // 16):
            v = pckv[j, pl.ds(k * 16, 16)]
            if srcv is not None:
                srcv[j, pl.ds(k * 16, 16)] = lax.shift_right_logical(v, 16)
            dstv[j, pl.ds(k * 16, 16)] = lax.bitwise_and(v, 0xFFFF)
        return carry

    lax.fori_loop(0, NB, unpack, 0)


def _make_counts():
    out_type = jax.ShapeDtypeStruct((3, NC, NPAD), _f32)
    scratch = [
        pltpu.VMEM_SHARED((NPAD,), _f32),
        pltpu.VMEM((NIB,), jnp.int32),
        pltpu.VMEM((NIB, B), jnp.int32),
        pltpu.VMEM((NB, B), jnp.int32),
        pltpu.VMEM((640,), _f32),
        pltpu.VMEM((B,), _f32),
        pltpu.SemaphoreType.DMA,
    ]

    def body(er, ea, ev, cnts, cacc, idxb, pckv, dstv, zct, ones, sem):
        c, s = _worker_ids(idxb)
        z16 = jnp.zeros((16,), _f32)
        one16 = jnp.ones((16,), _f32)

        def fillz(i, carry):
            zct[pl.ds(i * 16, 16)] = z16
            return carry

        lax.fori_loop(0, 640 // 16, fillz, 0)
        for i in range(B // 16):
            ones[pl.ds(i * 16, 16)] = one16

        for t, edg in enumerate((er, ea, ev)):
            pltpu.sync_copy(zct.at[pl.ds(0, RPS)], cacc.at[pl.ds(s * RPS, RPS)])
            plsc.subcore_barrier()
            _fetch_edges(edg, idxb, pckv, None, dstv, sem)

            def step(j, carry):
                pltpu.sync_copy(ones, cacc.at[dstv.at[j]], add=True)
                return carry

            lax.fori_loop(0, NB, step, 0)
            plsc.subcore_barrier()
            pltpu.sync_copy(cacc.at[pl.ds(s * RPS, RPS)],
                            cnts.at[t, c, pl.ds(s * RPS, RPS)])

    return pl.kernel(body, out_type=(out_type,), mesh=_mesh,
                     scratch_types=scratch, compiler_params=_sc_params)


def _make_segsum(with_counts):
    out_type = [jax.ShapeDtypeStruct((3, NC, NPAD, H), _f32)]
    scratch = [
        pltpu.VMEM_SHARED((NPAD, H), _f32),
        pltpu.VMEM((NIB,), jnp.int32),
        pltpu.VMEM((NIB, B), jnp.int32),
        pltpu.VMEM((NB, B), jnp.int32),
        pltpu.VMEM((NB, B), jnp.int32),
        [pltpu.VMEM((B, H), _f32)] * U,
        pltpu.VMEM((ZR, H), _f32),
        [pltpu.SemaphoreType.DMA] * U,
        [pltpu.SemaphoreType.DMA] * U,
    ]
    if with_counts:
        out_type.append(jax.ShapeDtypeStruct((3, NC, NPAD), _f32))
        scratch += [
            pltpu.VMEM_SHARED((NPAD,), _f32),
            pltpu.VMEM((640,), _f32),
            pltpu.VMEM((B,), _f32),
        ]

    def body(*refs):
        tr, ta, tv, er, ea, ev = refs[:6]
        if with_counts:
            sums, cnts = refs[6:8]
            (acc, idxb, pckv, srcv, dstv, rows, zac, gsems, ssems,
             cacc, zct, ones) = refs[8:]
        else:
            sums = refs[6]
            acc, idxb, pckv, srcv, dstv, rows, zac, gsems, ssems = refs[7:]
        c, s = _worker_ids(idxb)
        z16 = jnp.zeros((16,), _f32)

        def fill(i, carry):
            for q in range(H // 16):
                zac[i, pl.ds(q * 16, 16)] = z16
            return carry

        lax.fori_loop(0, ZR, fill, 0)
        if with_counts:
            one16 = jnp.ones((16,), _f32)

            def fillz(i, carry):
                zct[pl.ds(i * 16, 16)] = z16
                return carry

            lax.fori_loop(0, 640 // 16, fillz, 0)
            for i in range(B // 16):
                ones[pl.ds(i * 16, 16)] = one16

        for t, (tbl, edg) in enumerate(((tr, er), (ta, ea), (tv, ev))):
            pltpu.sync_copy(zac, acc.at[pl.ds(s * RPS, ZR)])
            pltpu.sync_copy(zac, acc.at[pl.ds(s * RPS + ZR, ZR)])
            if with_counts:
                pltpu.sync_copy(zct.at[pl.ds(0, RPS)],
                                cacc.at[pl.ds(s * RPS, RPS)])
            plsc.subcore_barrier()
            _fetch_edges(edg, idxb, pckv, srcv, dstv, gsems[0])

            def step(g, carry):
                j = g * U

                @pl.when(g > 0)
                def _():
                    for u in range(U):
                        pltpu.make_async_copy(
                            rows[u], acc.at[dstv.at[j - U + u]],
                            ssems[u]).wait()

                for u in range(U):
                    pltpu.async_copy(tbl.at[srcv.at[j + u]], rows[u], gsems[u])
                for u in range(U):
                    pltpu.make_async_copy(tbl.at[srcv.at[j + u]],
                                          rows[u], gsems[u]).wait()
                    pltpu.async_copy(rows[u], acc.at[dstv.at[j + u]],
                                     ssems[u], add=True)
                    if with_counts:
                        pltpu.sync_copy(ones, cacc.at[dstv.at[j + u]],
                                        add=True)
                return carry

            lax.fori_loop(0, NB // U, step, 0)
            for u in range(U):
                pltpu.make_async_copy(rows[u],
                                      acc.at[dstv.at[NB - U + u]],
                                      ssems[u]).wait()
            plsc.subcore_barrier()
            pltpu.sync_copy(acc.at[pl.ds(s * RPS, RPS)],
                            sums.at[t, c, pl.ds(s * RPS, RPS)])
            if with_counts:
                pltpu.sync_copy(cacc.at[pl.ds(s * RPS, RPS)],
                                cnts.at[t, c, pl.ds(s * RPS, RPS)])

    return pl.kernel(body, out_type=tuple(out_type), mesh=_mesh,
                     scratch_types=scratch, compiler_params=_sc_params)


_segsum_counts = _make_segsum(True)
_segsum_plain = _make_segsum(False)


R = 1000
GRID = N // R


def _dot(a, b):
    return jnp.dot(a, b, preferred_element_type=_f32)


def _project3(xu, xi, w_r, w_a, w_v):
    def body(xu_ref, xi_ref, wr_ref, wa_ref, wv_ref, tr_ref, ta_ref, tv_ref):
        tr_ref[...] = _dot(xu_ref[...], wr_ref[...])
        xir = xi_ref[...]
        ta_ref[...] = _dot(xir, wa_ref[...])
        tv_ref[...] = _dot(xir, wv_ref[...])

    return pl.pallas_call(
        body,
        grid=(GRID,),
        in_specs=[
            pl.BlockSpec((R, D), lambda i: (i, 0)),
            pl.BlockSpec((R, D), lambda i: (i, 0)),
            pl.BlockSpec((D, H), lambda i: (0, 0)),
            pl.BlockSpec((D, H), lambda i: (0, 0)),
            pl.BlockSpec((D, H), lambda i: (0, 0)),
        ],
        out_specs=[pl.BlockSpec((R, H), lambda i: (i, 0))] * 3,
        out_shape=[jax.ShapeDtypeStruct((N, H), _f32)] * 3,
    )(xu, xi, w_r, w_a, w_v)


def _combine1(sums, cnts, xu, xi, w1rr, w1ar, w1vr, b1r, b1a, b1v,
              w2rl, w2al, w2vl):
    def body(s_ref, c_ref, xu_ref, xi_ref, w1rr_ref, w1ar_ref, w1vr_ref,
             b1r_ref, b1a_ref, b1v_ref, w2rl_ref, w2al_ref, w2vl_ref,
             u1_ref, i1_ref, t2r_ref, t2a_ref, t2v_ref):
        sr = s_ref[0, 0] + s_ref[0, 1]
        sa = s_ref[1, 0] + s_ref[1, 1]
        sv = s_ref[2, 0] + s_ref[2, 1]
        cr = c_ref[0, 0] + c_ref[0, 1]
        ca = c_ref[1, 0] + c_ref[1, 1]
        cv = c_ref[2, 0] + c_ref[2, 1]
        item = (sr / jnp.maximum(cr, 1.0) + b1r_ref[...]
                + sa / jnp.maximum(ca, 1.0) + b1a_ref[...]
                + _dot(xi_ref[...], w1rr_ref[...] + w1ar_ref[...]))
        user = (sv / jnp.maximum(cv, 1.0) + b1v_ref[...]
                + _dot(xu_ref[...], w1vr_ref[...]))
        item = jnp.maximum(item, 0.0)
        user = jnp.maximum(user, 0.0)
        u1_ref[...] = user
        i1_ref[...] = item
        t2r_ref[...] = _dot(user, w2rl_ref[...])
        t2a_ref[...] = _dot(item, w2al_ref[...])
        t2v_ref[...] = _dot(item, w2vl_ref[...])

    return pl.pallas_call(
        body,
        grid=(GRID,),
        in_specs=[
            pl.BlockSpec((3, NC, R, H), lambda i: (0, 0, i, 0)),
            pl.BlockSpec((3, NC, R, 1), lambda i: (0, 0, i, 0)),
            pl.BlockSpec((R, D), lambda i: (i, 0)),
            pl.BlockSpec((R, D), lambda i: (i, 0)),
            pl.BlockSpec((D, H), lambda i: (0, 0)),
            pl.BlockSpec((D, H), lambda i: (0, 0)),
            pl.BlockSpec((D, H), lambda i: (0, 0)),
            pl.BlockSpec((1, H), lambda i: (0, 0)),
            pl.BlockSpec((1, H), lambda i: (0, 0)),
            pl.BlockSpec((1, H), lambda i: (0, 0)),
            pl.BlockSpec((H, H), lambda i: (0, 0)),
            pl.BlockSpec((H, H), lambda i: (0, 0)),
            pl.BlockSpec((H, H), lambda i: (0, 0)),
        ],
        out_specs=[pl.BlockSpec((R, H), lambda i: (i, 0))] * 5,
        out_shape=[jax.ShapeDtypeStruct((N, H), _f32)] * 5,
    )(sums, cnts, xu, xi, w1rr, w1ar, w1vr, b1r, b1a, b1v, w2rl, w2al, w2vl)


def _combine2(sums, cnts, u1, i1, w2rr, w2ar, w2vr, b2r, b2a, b2v):
    def body(s_ref, c_ref, u1_ref, i1_ref, w2rr_ref, w2ar_ref, w2vr_ref,
             b2r_ref, b2a_ref, b2v_ref, u2_ref, i2_ref):
        sr = s_ref[0, 0] + s_ref[0, 1]
        sa = s_ref[1, 0] + s_ref[1, 1]
        sv = s_ref[2, 0] + s_ref[2, 1]
        cr = c_ref[0, 0] + c_ref[0, 1]
        ca = c_ref[1, 0] + c_ref[1, 1]
        cv = c_ref[2, 0] + c_ref[2, 1]
        i2_ref[...] = (sr / jnp.maximum(cr, 1.0) + b2r_ref[...]
                       + sa / jnp.maximum(ca, 1.0) + b2a_ref[...]
                       + _dot(i1_ref[...], w2rr_ref[...] + w2ar_ref[...]))
        u2_ref[...] = (sv / jnp.maximum(cv, 1.0) + b2v_ref[...]
                       + _dot(u1_ref[...], w2vr_ref[...]))

    return pl.pallas_call(
        body,
        grid=(GRID,),
        in_specs=[
            pl.BlockSpec((3, NC, R, H), lambda i: (0, 0, i, 0)),
            pl.BlockSpec((3, NC, R, 1), lambda i: (0, 0, i, 0)),
            pl.BlockSpec((R, H), lambda i: (i, 0)),
            pl.BlockSpec((R, H), lambda i: (i, 0)),
            pl.BlockSpec((H, H), lambda i: (0, 0)),
            pl.BlockSpec((H, H), lambda i: (0, 0)),
            pl.BlockSpec((H, H), lambda i: (0, 0)),
            pl.BlockSpec((1, H), lambda i: (0, 0)),
            pl.BlockSpec((1, H), lambda i: (0, 0)),
            pl.BlockSpec((1, H), lambda i: (0, 0)),
        ],
        out_specs=[pl.BlockSpec((R, H), lambda i: (i, 0))] * 2,
        out_shape=[jax.ShapeDtypeStruct((N, H), _f32)] * 2,
    )(sums, cnts, u1, i1, w2rr, w2ar, w2vr, b2r, b2a, b2v)


def kernel(x_user, x_item, edge_reviews, edge_rev_reviews, edge_also_bought,
           W1r_l, b1r, W1r_r, W1v_l, b1v, W1v_r, W1a_l, b1a, W1a_r,
           W2r_l, b2r, W2r_r, W2v_l, b2v, W2v_r, W2a_l, b2a, W2a_r):
    def _prep(e):
        e = e.astype(jnp.int32)
        packed = ((e[0] << 16) | e[1]).reshape(NW * NB, B)
        pad = jnp.zeros((EROWS - NW * NB, B), jnp.int32)
        return jnp.concatenate([packed, pad], axis=0)

    er = _prep(edge_reviews)
    ev = _prep(edge_rev_reviews)
    ea = _prep(edge_also_bought)

    t1r, t1a, t1v = _project3(x_user, x_item, W1r_l, W1a_l, W1v_l)
    sums1, cnts = _segsum_counts(t1r, t1a, t1v, er, ea, ev)
    cnts = cnts.reshape(3, NC, NPAD, 1)
    u1, i1, t2r, t2a, t2v = _combine1(
        sums1, cnts, x_user, x_item, W1r_r, W1a_r, W1v_r,
        b1r.reshape(1, H), b1a.reshape(1, H), b1v.reshape(1, H),
        W2r_l, W2a_l, W2v_l)
    (sums2,) = _segsum_plain(t2r, t2a, t2v, er, ea, ev)
    u2, i2 = _combine2(
        sums2, cnts, u1, i1, W2r_r, W2a_r, W2v_r,
        b2r.reshape(1, H), b2a.reshape(1, H), b2v.reshape(1, H))
    return (u2, i2)


# speedup vs baseline: 2.7078x; 1.0606x over previous
import jax
import jax.numpy as jnp
from jax import lax
from jax.experimental import pallas as pl
from jax.experimental.pallas import tpu as pltpu
from jax.experimental.pallas import tpu_sc as plsc

N = 10000
D = 128
H = 64
E = 320000
NC = 2
NS = 16
NW = NC * NS
EPW = E // NW
B = 80
NB = EPW // B
NIB = 128
EROWS = 4096
NPAD = 10112
RPS = NPAD // NS
ZR = RPS // 2
U = 5

_f32 = jnp.float32
_mesh = plsc.VectorSubcoreMesh(core_axis_name="c", subcore_axis_name="s")
_sc_params = pltpu.CompilerParams(use_tc_tiling_on_sc=False)


def _worker_ids(idxb):
    c = lax.axis_index("c")
    s = lax.axis_index("s")
    w = c * NS + s
    lane = lax.iota(jnp.int32, 16)
    for k in range(NIB // 16):
        idxb[pl.ds(k * 16, 16)] = w * NB + k * 16 + lane
    return c, s


def _fetch_edges(edg, idxb, pckv, srcv, dstv, sem):
    pltpu.async_copy(edg.at[idxb], pckv, sem)
    pltpu.make_async_copy(edg.at[idxb], pckv, sem).wait()

    def unpack(j, carry):
        for k in range(B // 16):
            v = pckv[j, pl.ds(k * 16, 16)]
            if srcv is not None:
                srcv[j, pl.ds(k * 16, 16)] = lax.shift_right_logical(v, 16)
            dstv[j, pl.ds(k * 16, 16)] = lax.bitwise_and(v, 0xFFFF)
        return carry

    lax.fori_loop(0, NB, unpack, 0)


def _make_counts():
    out_type = jax.ShapeDtypeStruct((3, NC, NPAD), _f32)
    scratch = [
        pltpu.VMEM_SHARED((NPAD,), _f32),
        pltpu.VMEM((NIB,), jnp.int32),
        pltpu.VMEM((NIB, B), jnp.int32),
        pltpu.VMEM((NB, B), jnp.int32),
        pltpu.VMEM((640,), _f32),
        pltpu.VMEM((B,), _f32),
        pltpu.SemaphoreType.DMA,
    ]

    def body(er, ea, ev, cnts, cacc, idxb, pckv, dstv, zct, ones, sem):
        c, s = _worker_ids(idxb)
        z16 = jnp.zeros((16,), _f32)
        one16 = jnp.ones((16,), _f32)

        def fillz(i, carry):
            zct[pl.ds(i * 16, 16)] = z16
            return carry

        lax.fori_loop(0, 640 // 16, fillz, 0)
        for i in range(B // 16):
            ones[pl.ds(i * 16, 16)] = one16

        for t, edg in enumerate((er, ea, ev)):
            pltpu.sync_copy(zct.at[pl.ds(0, RPS)], cacc.at[pl.ds(s * RPS, RPS)])
            plsc.subcore_barrier()
            _fetch_edges(edg, idxb, pckv, None, dstv, sem)

            def step(j, carry):
                pltpu.sync_copy(ones, cacc.at[dstv.at[j]], add=True)
                return carry

            lax.fori_loop(0, NB, step, 0)
            plsc.subcore_barrier()
            pltpu.sync_copy(cacc.at[pl.ds(s * RPS, RPS)],
                            cnts.at[t, c, pl.ds(s * RPS, RPS)])

    return pl.kernel(body, out_type=(out_type,), mesh=_mesh,
                     scratch_types=scratch, compiler_params=_sc_params)


def _make_segsum(with_counts):
    out_type = [jax.ShapeDtypeStruct((3, NC, NPAD, H), _f32)]
    scratch = [
        pltpu.VMEM_SHARED((NPAD, H), _f32),
        pltpu.VMEM((NIB,), jnp.int32),
        pltpu.VMEM((NIB, B), jnp.int32),
        pltpu.VMEM((NB, B), jnp.int32),
        pltpu.VMEM((NB, B), jnp.int32),
        [pltpu.VMEM((B, H), _f32)] * U,
        pltpu.VMEM((ZR, H), _f32),
        [pltpu.SemaphoreType.DMA] * U,
        [pltpu.SemaphoreType.DMA] * U,
    ]
    if with_counts:
        out_type.append(jax.ShapeDtypeStruct((3, NC, NPAD), _f32))
        scratch += [
            pltpu.VMEM_SHARED((NPAD,), _f32),
            pltpu.VMEM((640,), _f32),
            pltpu.VMEM((B,), _f32),
        ]

    def body(*refs):
        tr, ta, tv, er, ea, ev = refs[:6]
        if with_counts:
            sums, cnts = refs[6:8]
            (acc, idxb, pckv, srcv, dstv, rows, zac, gsems, ssems,
             cacc, zct, ones) = refs[8:]
        else:
            sums = refs[6]
            acc, idxb, pckv, srcv, dstv, rows, zac, gsems, ssems = refs[7:]
        c, s = _worker_ids(idxb)
        z16 = jnp.zeros((16,), _f32)

        def fill(i, carry):
            for q in range(H // 16):
                zac[i, pl.ds(q * 16, 16)] = z16
            return carry

        lax.fori_loop(0, ZR, fill, 0)
        if with_counts:
            one16 = jnp.ones((16,), _f32)

            def fillz(i, carry):
                zct[pl.ds(i * 16, 16)] = z16
                return carry

            lax.fori_loop(0, 640 // 16, fillz, 0)
            for i in range(B // 16):
                ones[pl.ds(i * 16, 16)] = one16

        for t, (tbl, edg) in enumerate(((tr, er), (ta, ea), (tv, ev))):
            pltpu.sync_copy(zac, acc.at[pl.ds(s * RPS, ZR)])
            pltpu.sync_copy(zac, acc.at[pl.ds(s * RPS + ZR, ZR)])
            if with_counts:
                pltpu.sync_copy(zct.at[pl.ds(0, RPS)],
                                cacc.at[pl.ds(s * RPS, RPS)])
            plsc.subcore_barrier()
            _fetch_edges(edg, idxb, pckv, srcv, dstv, gsems[0])

            def step(g, carry):
                j = g * U

                @pl.when(g > 0)
                def _():
                    for u in range(U):
                        pltpu.make_async_copy(
                            rows[u], acc.at[dstv.at[j - U + u]],
                            ssems[u]).wait()

                for u in range(U):
                    pltpu.async_copy(tbl.at[srcv.at[j + u]], rows[u], gsems[u])
                for u in range(U):
                    pltpu.make_async_copy(tbl.at[srcv.at[j + u]],
                                          rows[u], gsems[u]).wait()
                    pltpu.async_copy(rows[u], acc.at[dstv.at[j + u]],
                                     ssems[u], add=True)
                    if with_counts:
                        pltpu.sync_copy(ones, cacc.at[dstv.at[j + u]],
                                        add=True)
                return carry

            lax.fori_loop(0, NB // U, step, 0)
            for u in range(U):
                pltpu.make_async_copy(rows[u],
                                      acc.at[dstv.at[NB - U + u]],
                                      ssems[u]).wait()
            plsc.subcore_barrier()
            pltpu.sync_copy(acc.at[pl.ds(s * RPS, RPS)],
                            sums.at[t, c, pl.ds(s * RPS, RPS)])
            if with_counts:
                pltpu.sync_copy(cacc.at[pl.ds(s * RPS, RPS)],
                                cnts.at[t, c, pl.ds(s * RPS, RPS)])

    return pl.kernel(body, out_type=tuple(out_type), mesh=_mesh,
                     scratch_types=scratch, compiler_params=_sc_params)


_segsum_counts = _make_segsum(True)
_segsum_plain = _make_segsum(False)


R = 2000
GRID = N // R


def _dot(a, b):
    return jnp.dot(a, b, preferred_element_type=_f32)


EBR = E // 128
EBRP = EROWS * B // 128
EBB = EBRP // GRID


def _project3(xu, xi, w_r, w_a, w_v, e_r, e_v, e_a):
    def body(xu_ref, xi_ref, wr_ref, wa_ref, wv_ref, er_ref, ev_ref, ea_ref,
             tr_ref, ta_ref, tv_ref, pr_ref, pv_ref, pa_ref):
        tr_ref[...] = _dot(xu_ref[...], wr_ref[...])
        xir = xi_ref[...]
        ta_ref[...] = _dot(xir, wa_ref[...])
        tv_ref[...] = _dot(xir, wv_ref[...])
        for e_ref, p_ref in ((er_ref, pr_ref), (ev_ref, pv_ref),
                             (ea_ref, pa_ref)):
            ee = e_ref[...]
            p_ref[...] = lax.shift_left(ee[0], 16) | ee[1]

    return pl.pallas_call(
        body,
        grid=(GRID,),
        in_specs=[
            pl.BlockSpec((R, D), lambda i: (i, 0)),
            pl.BlockSpec((R, D), lambda i: (i, 0)),
            pl.BlockSpec((D, H), lambda i: (0, 0)),
            pl.BlockSpec((D, H), lambda i: (0, 0)),
            pl.BlockSpec((D, H), lambda i: (0, 0)),
            pl.BlockSpec((2, EBB, 128), lambda i: (0, i, 0)),
            pl.BlockSpec((2, EBB, 128), lambda i: (0, i, 0)),
            pl.BlockSpec((2, EBB, 128), lambda i: (0, i, 0)),
        ],
        out_specs=([pl.BlockSpec((R, H), lambda i: (i, 0))] * 3
                   + [pl.BlockSpec((EBB, 128), lambda i: (i, 0))] * 3),
        out_shape=([jax.ShapeDtypeStruct((N, H), _f32)] * 3
                   + [jax.ShapeDtypeStruct((EBRP, 128), jnp.int32)] * 3),
    )(xu, xi, w_r, w_a, w_v, e_r, e_v, e_a)


def _combine1(sums, cnts, xu, xi, w1rr, w1ar, w1vr, b1r, b1a, b1v,
              w2rl, w2al, w2vl):
    def body(s_ref, c_ref, xu_ref, xi_ref, w1rr_ref, w1ar_ref, w1vr_ref,
             b1r_ref, b1a_ref, b1v_ref, w2rl_ref, w2al_ref, w2vl_ref,
             u1_ref, i1_ref, t2r_ref, t2a_ref, t2v_ref):
        sr = s_ref[0, 0] + s_ref[0, 1]
        sa = s_ref[1, 0] + s_ref[1, 1]
        sv = s_ref[2, 0] + s_ref[2, 1]
        cr = c_ref[0, 0] + c_ref[0, 1]
        ca = c_ref[1, 0] + c_ref[1, 1]
        cv = c_ref[2, 0] + c_ref[2, 1]
        item = (sr / jnp.maximum(cr, 1.0) + b1r_ref[...]
                + sa / jnp.maximum(ca, 1.0) + b1a_ref[...]
                + _dot(xi_ref[...], w1rr_ref[...] + w1ar_ref[...]))
        user = (sv / jnp.maximum(cv, 1.0) + b1v_ref[...]
                + _dot(xu_ref[...], w1vr_ref[...]))
        item = jnp.maximum(item, 0.0)
        user = jnp.maximum(user, 0.0)
        u1_ref[...] = user
        i1_ref[...] = item
        t2r_ref[...] = _dot(user, w2rl_ref[...])
        t2a_ref[...] = _dot(item, w2al_ref[...])
        t2v_ref[...] = _dot(item, w2vl_ref[...])

    return pl.pallas_call(
        body,
        grid=(GRID,),
        in_specs=[
            pl.BlockSpec((3, NC, R, H), lambda i: (0, 0, i, 0)),
            pl.BlockSpec((3, NC, R, 1), lambda i: (0, 0, i, 0)),
            pl.BlockSpec((R, D), lambda i: (i, 0)),
            pl.BlockSpec((R, D), lambda i: (i, 0)),
            pl.BlockSpec((D, H), lambda i: (0, 0)),
            pl.BlockSpec((D, H), lambda i: (0, 0)),
            pl.BlockSpec((D, H), lambda i: (0, 0)),
            pl.BlockSpec((1, H), lambda i: (0, 0)),
            pl.BlockSpec((1, H), lambda i: (0, 0)),
            pl.BlockSpec((1, H), lambda i: (0, 0)),
            pl.BlockSpec((H, H), lambda i: (0, 0)),
            pl.BlockSpec((H, H), lambda i: (0, 0)),
            pl.BlockSpec((H, H), lambda i: (0, 0)),
        ],
        out_specs=[pl.BlockSpec((R, H), lambda i: (i, 0))] * 5,
        out_shape=[jax.ShapeDtypeStruct((N, H), _f32)] * 5,
    )(sums, cnts, xu, xi, w1rr, w1ar, w1vr, b1r, b1a, b1v, w2rl, w2al, w2vl)


def _combine2(sums, cnts, u1, i1, w2rr, w2ar, w2vr, b2r, b2a, b2v):
    def body(s_ref, c_ref, u1_ref, i1_ref, w2rr_ref, w2ar_ref, w2vr_ref,
             b2r_ref, b2a_ref, b2v_ref, u2_ref, i2_ref):
        sr = s_ref[0, 0] + s_ref[0, 1]
        sa = s_ref[1, 0] + s_ref[1, 1]
        sv = s_ref[2, 0] + s_ref[2, 1]
        cr = c_ref[0, 0] + c_ref[0, 1]
        ca = c_ref[1, 0] + c_ref[1, 1]
        cv = c_ref[2, 0] + c_ref[2, 1]
        i2_ref[...] = (sr / jnp.maximum(cr, 1.0) + b2r_ref[...]
                       + sa / jnp.maximum(ca, 1.0) + b2a_ref[...]
                       + _dot(i1_ref[...], w2rr_ref[...] + w2ar_ref[...]))
        u2_ref[...] = (sv / jnp.maximum(cv, 1.0) + b2v_ref[...]
                       + _dot(u1_ref[...], w2vr_ref[...]))

    return pl.pallas_call(
        body,
        grid=(GRID,),
        in_specs=[
            pl.BlockSpec((3, NC, R, H), lambda i: (0, 0, i, 0)),
            pl.BlockSpec((3, NC, R, 1), lambda i: (0, 0, i, 0)),
            pl.BlockSpec((R, H), lambda i: (i, 0)),
            pl.BlockSpec((R, H), lambda i: (i, 0)),
            pl.BlockSpec((H, H), lambda i: (0, 0)),
            pl.BlockSpec((H, H), lambda i: (0, 0)),
            pl.BlockSpec((H, H), lambda i: (0, 0)),
            pl.BlockSpec((1, H), lambda i: (0, 0)),
            pl.BlockSpec((1, H), lambda i: (0, 0)),
            pl.BlockSpec((1, H), lambda i: (0, 0)),
        ],
        out_specs=[pl.BlockSpec((R, H), lambda i: (i, 0))] * 2,
        out_shape=[jax.ShapeDtypeStruct((N, H), _f32)] * 2,
    )(sums, cnts, u1, i1, w2rr, w2ar, w2vr, b2r, b2a, b2v)


def kernel(x_user, x_item, edge_reviews, edge_rev_reviews, edge_also_bought,
           W1r_l, b1r, W1r_r, W1v_l, b1v, W1v_r, W1a_l, b1a, W1a_r,
           W2r_l, b2r, W2r_r, W2v_l, b2v, W2v_r, W2a_l, b2a, W2a_r):
    def _eprep(e):
        return e.astype(jnp.int32).reshape(2, EBR, 128)

    t1r, t1a, t1v, er, ev, ea = _project3(
        x_user, x_item, W1r_l, W1a_l, W1v_l,
        _eprep(edge_reviews), _eprep(edge_rev_reviews),
        _eprep(edge_also_bought))
    er, ev, ea = (e.reshape(EROWS, B) for e in (er, ev, ea))
    sums1, cnts = _segsum_counts(t1r, t1a, t1v, er, ea, ev)
    cnts = cnts.reshape(3, NC, NPAD, 1)
    u1, i1, t2r, t2a, t2v = _combine1(
        sums1, cnts, x_user, x_item, W1r_r, W1a_r, W1v_r,
        b1r.reshape(1, H), b1a.reshape(1, H), b1v.reshape(1, H),
        W2r_l, W2a_l, W2v_l)
    (sums2,) = _segsum_plain(t2r, t2a, t2v, er, ea, ev)
    u2, i2 = _combine2(
        sums2, cnts, u1, i1, W2r_r, W2a_r, W2v_r,
        b2r.reshape(1, H), b2a.reshape(1, H), b2v.reshape(1, H))
    return (u2, i2)
